# Initial kernel scaffold; baseline (speedup 1.0000x reference)
import jax
import jax.numpy as jnp
from jax.experimental import pallas as pl


def kernel(x, edge_index, edge_type, W1_rel, W1_root, b1, g1, be1, W2_rel, W2_root, b2, g2, be2, W3_rel, W3_root, b3):
    raise NotImplementedError("write your pallas kernel here")


# speedup vs baseline: 15.4440x; 15.4440x over previous
import jax
import jax.numpy as jnp
from jax import lax
from jax.experimental import pallas as pl
from jax.experimental.pallas import tpu as pltpu
from jax.experimental.pallas import tpu_sc as plsc

NC, NS = 2, 16
NW = NC * NS
CHUNK = 128
RPAD = 16
ETPAD = 15


def _sc_mesh():
    return plsc.VectorSubcoreMesh(
        core_axis_name="c", subcore_axis_name="s",
        num_cores=NC, num_subcores=NS)


def _bcast_lane(v, lane):
    return jnp.take(v, jnp.full((16,), lane, jnp.int32))


def _make_prep(N, R, CPW):
    NPS = N // NS
    SR = 125
    mesh = _sc_mesh()

    def body(src3, et3, dst3, g3, w3,
             cnt_sp, srcb, etb, dstb, gob, wob, invbuf, oh, ivrows, sem):
        cid = lax.axis_index("c")
        sid = lax.axis_index("s")
        wid = sid * NC + cid
        iota16 = lax.iota(jnp.int32, 16)
        zeros16 = jnp.zeros((16,), jnp.float32)

        def zrow(r, _):
            invbuf[r, :] = zeros16
            return 0
        lax.fori_loop(0, SR, zrow, 0)
        for t in range(NPS // SR):
            pltpu.sync_copy(invbuf,
                            cnt_sp.at[pl.ds(sid * NPS + t * SR, SR)])
        plsc.subcore_barrier()

        for blk in range(2):
            wrow = sid * 2 + blk
            pltpu.sync_copy(dst3.at[wrow], dstb)
            pltpu.sync_copy(et3.at[wrow], etb)

            def cbody(c, _):
                def jgrp(j16, _):
                    s0 = pl.multiple_of(j16 * 16, 16)
                    etv = etb[c, pl.ds(s0, 16)]
                    for l in range(16):
                        ebc = _bcast_lane(etv, l)
                        oh[j16 * 16 + l, :] = jnp.where(
                            iota16 == ebc, 1.0, 0.0)
                    return 0
                lax.fori_loop(0, 8, jgrp, 0)
                pltpu.sync_copy(oh, cnt_sp.at[dstb.at[c]], add=True)
                return 0
            lax.fori_loop(0, CPW, cbody, 0)
        plsc.subcore_barrier()

        for t in range(NPS // SR):
            base = sid * NPS + t * SR
            pltpu.sync_copy(cnt_sp.at[pl.ds(base, SR)], invbuf)

            def irow(r, _):
                invbuf[r, :] = 1.0 / jnp.maximum(invbuf[r, :], 1.0)
                return 0
            lax.fori_loop(0, SR, irow, 0)
            pltpu.sync_copy(invbuf, cnt_sp.at[pl.ds(base, SR)])
        plsc.subcore_barrier()

        pltpu.sync_copy(src3.at[wid], srcb)
        pltpu.sync_copy(et3.at[wid], etb)
        pltpu.sync_copy(dst3.at[wid], dstb)

        def cbody2(c, _):
            pltpu.async_copy(cnt_sp.at[dstb.at[c]], ivrows, sem).wait()

            def jgrp(j16, _):
                s0 = pl.multiple_of(j16 * 16, 16)
                etv = etb[c, pl.ds(s0, 16)]
                srcv = srcb[c, pl.ds(s0, 16)]
                valid = etv < R
                gob[c, pl.ds(s0, 16)] = jnp.where(valid, etv * N + srcv, 0)
                wacc = zeros16
                for l in range(16):
                    row16 = ivrows[j16 * 16 + l, :]
                    ebc = _bcast_lane(etv, l)
                    wbc = jnp.take(row16, ebc)
                    wacc = jnp.where(iota16 == l, wbc, wacc)
                wob[c, pl.ds(s0, 16)] = jnp.where(valid, wacc, 0.0)
                return 0
            lax.fori_loop(0, 8, jgrp, 0)
            return 0
        lax.fori_loop(0, CPW, cbody2, 0)
        pltpu.sync_copy(gob, g3.at[wid])
        pltpu.sync_copy(wob, w3.at[wid])

    return pl.kernel(
        body,
        out_type=(jax.ShapeDtypeStruct((NW, CPW, CHUNK), jnp.int32),
                  jax.ShapeDtypeStruct((NW, CPW, CHUNK), jnp.float32)),
        mesh=mesh,
        scratch_types=[
            pltpu.VMEM_SHARED((N, RPAD), jnp.float32),
            pltpu.VMEM((CPW, CHUNK), jnp.int32),
            pltpu.VMEM((CPW, CHUNK), jnp.int32),
            pltpu.VMEM((CPW, CHUNK), jnp.int32),
            pltpu.VMEM((CPW, CHUNK), jnp.int32),
            pltpu.VMEM((CPW, CHUNK), jnp.float32),
            pltpu.VMEM((SR, RPAD), jnp.float32),
            pltpu.VMEM((CHUNK, RPAD), jnp.float32),
            pltpu.VMEM((CHUNK, RPAD), jnp.float32),
            pltpu.SemaphoreType.DMA,
        ],
    )


def _make_edge_pass(N, D, CPW):
    BR = (N // NS) // 8 * 8
    TAIL = N - NS * BR
    ZR = 48
    BB = 16
    assert BR % ZR == 0 and TAIL % 8 == 0 and CPW % BB == 0
    mesh = _sc_mesh()

    def body(T, dst3, g3, w3, out, acc_sp, gb, db, wb, rows, zb, sem):
        cid = lax.axis_index("c")
        sid = lax.axis_index("s")
        wid = sid * NC + cid

        def zrow(r, _):
            for k in range(D // 16):
                zb[r, pl.ds(k * 16, 16)] = jnp.zeros((16,), jnp.float32)
            return 0
        lax.fori_loop(0, ZR, zrow, 0)
        for t in range(BR // ZR):
            pltpu.sync_copy(zb, acc_sp.at[pl.ds(sid * BR + t * ZR, ZR)])
        if TAIL:
            @pl.when(sid == 0)
            def _():
                pltpu.sync_copy(zb.at[pl.ds(0, TAIL)],
                                acc_sp.at[pl.ds(NS * BR, TAIL)])
        plsc.subcore_barrier()

        def bbody(b, _):
            pltpu.sync_copy(g3.at[wid, pl.ds(b * BB, BB)], gb)
            pltpu.sync_copy(w3.at[wid, pl.ds(b * BB, BB)], wb)
            pltpu.sync_copy(dst3.at[wid, pl.ds(b * BB, BB)], db)

            def cbody(c, _):
                pltpu.async_copy(T.at[gb.at[c]], rows, sem).wait()

                def jgrp(j16, _):
                    s0 = pl.multiple_of(j16 * 16, 16)
                    wv16 = wb[c, pl.ds(s0, 16)]
                    for l in range(16):
                        bc = _bcast_lane(wv16, l)
                        j = j16 * 16 + l
                        for k in range(D // 16):
                            sl = pl.ds(k * 16, 16)
                            rows[j, sl] = rows[j, sl] * bc
                    return 0
                lax.fori_loop(0, 8, jgrp, 0)
                pltpu.sync_copy(rows, acc_sp.at[db.at[c]], add=True)
                return 0
            lax.fori_loop(0, BB, cbody, 0)
            return 0
        lax.fori_loop(0, CPW // BB, bbody, 0)
        plsc.subcore_barrier()

        pltpu.sync_copy(acc_sp.at[pl.ds(sid * BR, BR)],
                        out.at[cid, pl.ds(sid * BR, BR)])
        if TAIL:
            @pl.when(sid == 0)
            def _():
                pltpu.sync_copy(acc_sp.at[pl.ds(NS * BR, TAIL)],
                                out.at[cid, pl.ds(NS * BR, TAIL)])

    return pl.kernel(
        body,
        out_type=jax.ShapeDtypeStruct((NC, N, D), jnp.float32),
        mesh=mesh,
        scratch_types=[
            pltpu.VMEM_SHARED((N, D), jnp.float32),
            pltpu.VMEM((BB, CHUNK), jnp.int32),
            pltpu.VMEM((BB, CHUNK), jnp.int32),
            pltpu.VMEM((BB, CHUNK), jnp.float32),
            pltpu.VMEM((CHUNK, D), jnp.float32),
            pltpu.VMEM((ZR, D), jnp.float32),
            pltpu.SemaphoreType.DMA,
        ],
    )


def _tt_body(h_ref, w_ref, o_ref):
    o_ref[0] = jnp.dot(h_ref[...], w_ref[0],
                       preferred_element_type=jnp.float32)


def _ttable(h, W_rel):
    N, Din = h.shape
    Rr, _, Dout = W_rel.shape
    BN = 2000
    T = pl.pallas_call(
        _tt_body,
        grid=(Rr, N // BN),
        in_specs=[pl.BlockSpec((BN, Din), lambda r, nb: (nb, 0)),
                  pl.BlockSpec((1, Din, Dout), lambda r, nb: (r, 0, 0))],
        out_specs=pl.BlockSpec((1, BN, Dout), lambda r, nb: (r, nb, 0)),
        out_shape=jax.ShapeDtypeStruct((Rr, N, Dout), jnp.float32),
    )(h, W_rel)
    return T.reshape(Rr * N, Dout)


def _post_bn_body(h_ref, w_ref, b_ref, acc_ref, g_ref, be_ref, o_ref):
    y = jnp.dot(h_ref[...], w_ref[...], preferred_element_type=jnp.float32)
    y = y + b_ref[...][None, :] + acc_ref[0] + acc_ref[1]
    m = jnp.mean(y, axis=0, keepdims=True)
    v = jnp.mean(jnp.square(y - m), axis=0, keepdims=True)
    y = (y - m) * lax.rsqrt(v + 1e-5) * g_ref[...][None, :] + be_ref[...][None, :]
    o_ref[...] = jnp.maximum(y, 0.0)


def _post_bn(h, W_root, b, acc, gmm, bet):
    N = h.shape[0]
    Dout = W_root.shape[1]
    return pl.pallas_call(
        _post_bn_body,
        out_shape=jax.ShapeDtypeStruct((N, Dout), jnp.float32),
    )(h, W_root, b, acc, gmm, bet)


def _post_final_body(h_ref, w_ref, b_ref, acc_ref, o_ref):
    y = jnp.dot(h_ref[...], w_ref[...], preferred_element_type=jnp.float32)
    o_ref[...] = y + b_ref[...][None, :] + acc_ref[0] + acc_ref[1]


def _post_final(h, W_root, b, acc):
    N = h.shape[0]
    Dout = W_root.shape[1]
    return pl.pallas_call(
        _post_final_body,
        out_shape=jax.ShapeDtypeStruct((N, Dout), jnp.float32),
    )(h, W_root, b, acc)


def kernel(x, edge_index, edge_type, W1_rel, W1_root, b1, g1, be1,
           W2_rel, W2_root, b2, g2, be2, W3_rel, W3_root, b3):
    N, _ = x.shape
    R = W1_rel.shape[0]
    E = edge_type.shape[0]
    CPW = -(-E // (NW * CHUNK))
    CPW = -(-CPW // 16) * 16
    EP = NW * CPW * CHUNK
    pad = EP - E

    src = edge_index[0]
    dst = edge_index[1]
    zpad = jnp.zeros((pad,), jnp.int32)
    src3 = jnp.concatenate([src, zpad]).reshape(NW, CPW, CHUNK)
    dst3 = jnp.concatenate([dst, zpad]).reshape(NW, CPW, CHUNK)
    et3 = jnp.concatenate(
        [edge_type, jnp.full((pad,), ETPAD, jnp.int32)]).reshape(NW, CPW, CHUNK)

    g3, w3 = _make_prep(N, R, CPW)(src3, et3, dst3)

    def layer(h, W_rel, W_root, b, post):
        Dout = W_rel.shape[2]
        T = _ttable(h, W_rel)
        acc = _make_edge_pass(N, Dout, CPW)(T, dst3, g3, w3)
        return post(h, W_root, b, acc)

    h = layer(x, W1_rel, W1_root, b1,
              lambda h_, w_, b_, a_: _post_bn(h_, w_, b_, a_, g1, be1))
    h = layer(h, W2_rel, W2_root, b2,
              lambda h_, w_, b_, a_: _post_bn(h_, w_, b_, a_, g2, be2))

    DP = 128
    W3p = jnp.pad(W3_rel, ((0, 0), (0, 0), (0, DP - W3_rel.shape[2])))
    W3rootp = jnp.pad(W3_root, ((0, 0), (0, DP - W3_root.shape[1])))
    b3p = jnp.pad(b3, (0, DP - b3.shape[0]))
    out16 = layer(h, W3p, W3rootp, b3p, _post_final)
    return out16[:, :W3_rel.shape[2]]


# speedup vs baseline: 17.7406x; 1.1487x over previous
import jax
import jax.numpy as jnp
from jax import lax
from jax.experimental import pallas as pl
from jax.experimental.pallas import tpu as pltpu
from jax.experimental.pallas import tpu_sc as plsc

NC, NS = 2, 16
NW = NC * NS
CHUNK = 128
RPAD = 16
ETPAD = 15


def _sc_mesh():
    return plsc.VectorSubcoreMesh(
        core_axis_name="c", subcore_axis_name="s",
        num_cores=NC, num_subcores=NS)


def _bcast_lane(v, lane):
    return jnp.take(v, jnp.full((16,), lane, jnp.int32))


def _make_prep(N, R, CPW):
    NPS = N // NS
    SR = 125
    mesh = _sc_mesh()

    def body(src3, et3, dst3, g3, w3,
             cnt_sp, srcb, etb, dstb, gob, wob, invbuf, oh, ivrows, sem):
        cid = lax.axis_index("c")
        sid = lax.axis_index("s")
        wid = sid * NC + cid
        iota16 = lax.iota(jnp.int32, 16)
        zeros16 = jnp.zeros((16,), jnp.float32)

        def zrow(r, _):
            invbuf[r, :] = zeros16
            return 0
        lax.fori_loop(0, SR, zrow, 0)
        for t in range(NPS // SR):
            pltpu.sync_copy(invbuf,
                            cnt_sp.at[pl.ds(sid * NPS + t * SR, SR)])
        plsc.subcore_barrier()

        for blk in range(2):
            wrow = sid * 2 + blk
            pltpu.sync_copy(dst3.at[wrow], dstb)
            pltpu.sync_copy(et3.at[wrow], etb)

            def cbody(c, _):
                def jgrp(j16, _):
                    s0 = pl.multiple_of(j16 * 16, 16)
                    etv = etb[c, pl.ds(s0, 16)]
                    for l in range(16):
                        ebc = _bcast_lane(etv, l)
                        oh[j16 * 16 + l, :] = jnp.where(
                            iota16 == ebc, 1.0, 0.0)
                    return 0
                lax.fori_loop(0, 8, jgrp, 0)
                pltpu.sync_copy(oh, cnt_sp.at[dstb.at[c]], add=True)
                return 0
            lax.fori_loop(0, CPW, cbody, 0)
        plsc.subcore_barrier()

        for t in range(NPS // SR):
            base = sid * NPS + t * SR
            pltpu.sync_copy(cnt_sp.at[pl.ds(base, SR)], invbuf)

            def irow(r, _):
                invbuf[r, :] = 1.0 / jnp.maximum(invbuf[r, :], 1.0)
                return 0
            lax.fori_loop(0, SR, irow, 0)
            pltpu.sync_copy(invbuf, cnt_sp.at[pl.ds(base, SR)])
        plsc.subcore_barrier()

        pltpu.sync_copy(src3.at[wid], srcb)
        pltpu.sync_copy(et3.at[wid], etb)
        pltpu.sync_copy(dst3.at[wid], dstb)

        def cbody2(c, _):
            pltpu.async_copy(cnt_sp.at[dstb.at[c]], ivrows, sem).wait()

            def jgrp(j16, _):
                s0 = pl.multiple_of(j16 * 16, 16)
                etv = etb[c, pl.ds(s0, 16)]
                srcv = srcb[c, pl.ds(s0, 16)]
                valid = etv < R
                gob[c, pl.ds(s0, 16)] = jnp.where(valid, etv * N + srcv, 0)
                wacc = zeros16
                for l in range(16):
                    row16 = ivrows[j16 * 16 + l, :]
                    ebc = _bcast_lane(etv, l)
                    wbc = jnp.take(row16, ebc)
                    wacc = jnp.where(iota16 == l, wbc, wacc)
                wob[c, pl.ds(s0, 16)] = jnp.where(valid, wacc, 0.0)
                return 0
            lax.fori_loop(0, 8, jgrp, 0)
            return 0
        lax.fori_loop(0, CPW, cbody2, 0)
        pltpu.sync_copy(gob, g3.at[wid])
        pltpu.sync_copy(wob, w3.at[wid])

    return pl.kernel(
        body,
        out_type=(jax.ShapeDtypeStruct((NW, CPW, CHUNK), jnp.int32),
                  jax.ShapeDtypeStruct((NW, CPW, CHUNK), jnp.float32)),
        mesh=mesh,
        scratch_types=[
            pltpu.VMEM_SHARED((N, RPAD), jnp.float32),
            pltpu.VMEM((CPW, CHUNK), jnp.int32),
            pltpu.VMEM((CPW, CHUNK), jnp.int32),
            pltpu.VMEM((CPW, CHUNK), jnp.int32),
            pltpu.VMEM((CPW, CHUNK), jnp.int32),
            pltpu.VMEM((CPW, CHUNK), jnp.float32),
            pltpu.VMEM((SR, RPAD), jnp.float32),
            pltpu.VMEM((CHUNK, RPAD), jnp.float32),
            pltpu.VMEM((CHUNK, RPAD), jnp.float32),
            pltpu.SemaphoreType.DMA,
        ],
    )


def _make_edge_pass(N, D, CPW):
    BR = (N // NS) // 8 * 8
    TAIL = N - NS * BR
    ZR = 48
    BB = 16
    assert BR % ZR == 0 and TAIL % 8 == 0 and CPW % BB == 0
    mesh = _sc_mesh()

    def body(T, dst3, g3, w3, out, acc_sp,
             gb, db, wb, rows0, rows1, zb, gsem0, gsem1, ssem0, ssem1):
        cid = lax.axis_index("c")
        sid = lax.axis_index("s")
        wid = sid * NC + cid

        def zrow(r, _):
            for k in range(D // 16):
                zb[r, pl.ds(k * 16, 16)] = jnp.zeros((16,), jnp.float32)
            return 0
        lax.fori_loop(0, ZR, zrow, 0)
        for t in range(BR // ZR):
            pltpu.sync_copy(zb, acc_sp.at[pl.ds(sid * BR + t * ZR, ZR)])
        if TAIL:
            @pl.when(sid == 0)
            def _():
                pltpu.sync_copy(zb.at[pl.ds(0, TAIL)],
                                acc_sp.at[pl.ds(NS * BR, TAIL)])
        plsc.subcore_barrier()

        def scale(rows_ref, c):
            def jb(j, _):
                g16 = (j // 16) * 16
                wv16 = wb[c, pl.ds(pl.multiple_of(g16, 16), 16)]
                bc = _bcast_lane(wv16, j - g16)
                for k in range(D // 16):
                    sl = pl.ds(k * 16, 16)
                    rows_ref[j, sl] = rows_ref[j, sl] * bc
                return 0
            lax.fori_loop(0, CHUNK, jb, 0)

        def wait_g(rows_ref, gsem):
            pltpu.make_async_copy(T.at[gb.at[0]], rows_ref, gsem).wait()

        def wait_s(rows_ref, ssem):
            pltpu.make_async_copy(rows_ref, acc_sp.at[db.at[0]], ssem).wait()

        def bbody(b, _):
            pltpu.sync_copy(g3.at[wid, pl.ds(b * BB, BB)], gb)
            pltpu.sync_copy(w3.at[wid, pl.ds(b * BB, BB)], wb)
            pltpu.sync_copy(dst3.at[wid, pl.ds(b * BB, BB)], db)
            pltpu.async_copy(T.at[gb.at[0]], rows0, gsem0)

            def pbody(p, _):
                pltpu.async_copy(T.at[gb.at[2 * p + 1]], rows1, gsem1)
                wait_g(rows0, gsem0)
                scale(rows0, 2 * p)
                pltpu.sync_copy(rows0, acc_sp.at[db.at[2 * p]], add=True)
                pltpu.async_copy(T.at[gb.at[2 * p + 2]], rows0, gsem0)
                wait_g(rows1, gsem1)
                scale(rows1, 2 * p + 1)
                pltpu.sync_copy(rows1, acc_sp.at[db.at[2 * p + 1]], add=True)
                return 0
            lax.fori_loop(0, BB // 2 - 1, pbody, 0)
            pltpu.async_copy(T.at[gb.at[BB - 1]], rows1, gsem1)
            wait_g(rows0, gsem0)
            scale(rows0, BB - 2)
            pltpu.sync_copy(rows0, acc_sp.at[db.at[BB - 2]], add=True)
            wait_g(rows1, gsem1)
            scale(rows1, BB - 1)
            pltpu.sync_copy(rows1, acc_sp.at[db.at[BB - 1]], add=True)
            return 0
        lax.fori_loop(0, CPW // BB, bbody, 0)
        plsc.subcore_barrier()

        pltpu.sync_copy(acc_sp.at[pl.ds(sid * BR, BR)],
                        out.at[cid, pl.ds(sid * BR, BR)])
        if TAIL:
            @pl.when(sid == 0)
            def _():
                pltpu.sync_copy(acc_sp.at[pl.ds(NS * BR, TAIL)],
                                out.at[cid, pl.ds(NS * BR, TAIL)])

    return pl.kernel(
        body,
        out_type=jax.ShapeDtypeStruct((NC, N, D), jnp.float32),
        mesh=mesh,
        scratch_types=[
            pltpu.VMEM_SHARED((N, D), jnp.float32),
            pltpu.VMEM((BB, CHUNK), jnp.int32),
            pltpu.VMEM((BB, CHUNK), jnp.int32),
            pltpu.VMEM((BB, CHUNK), jnp.float32),
            pltpu.VMEM((CHUNK, D), jnp.float32),
            pltpu.VMEM((CHUNK, D), jnp.float32),
            pltpu.VMEM((ZR, D), jnp.float32),
            pltpu.SemaphoreType.DMA,
            pltpu.SemaphoreType.DMA,
            pltpu.SemaphoreType.DMA,
            pltpu.SemaphoreType.DMA,
        ],
    )


def _tt_body(h_ref, w_ref, o_ref):
    o_ref[0] = jnp.dot(h_ref[...], w_ref[0],
                       preferred_element_type=jnp.float32)


def _ttable(h, W_rel):
    N, Din = h.shape
    Rr, _, Dout = W_rel.shape
    BN = 2000
    T = pl.pallas_call(
        _tt_body,
        grid=(Rr, N // BN),
        in_specs=[pl.BlockSpec((BN, Din), lambda r, nb: (nb, 0)),
                  pl.BlockSpec((1, Din, Dout), lambda r, nb: (r, 0, 0))],
        out_specs=pl.BlockSpec((1, BN, Dout), lambda r, nb: (r, nb, 0)),
        out_shape=jax.ShapeDtypeStruct((Rr, N, Dout), jnp.float32),
    )(h, W_rel)
    return T.reshape(Rr * N, Dout)


def _post_bn_body(h_ref, w_ref, b_ref, acc_ref, g_ref, be_ref, o_ref):
    y = jnp.dot(h_ref[...], w_ref[...], preferred_element_type=jnp.float32)
    y = y + b_ref[...][None, :] + acc_ref[0] + acc_ref[1]
    m = jnp.mean(y, axis=0, keepdims=True)
    v = jnp.mean(jnp.square(y - m), axis=0, keepdims=True)
    y = (y - m) * lax.rsqrt(v + 1e-5) * g_ref[...][None, :] + be_ref[...][None, :]
    o_ref[...] = jnp.maximum(y, 0.0)


def _post_bn(h, W_root, b, acc, gmm, bet):
    N = h.shape[0]
    Dout = W_root.shape[1]
    return pl.pallas_call(
        _post_bn_body,
        out_shape=jax.ShapeDtypeStruct((N, Dout), jnp.float32),
    )(h, W_root, b, acc, gmm, bet)


def _post_final_body(h_ref, w_ref, b_ref, acc_ref, o_ref):
    y = jnp.dot(h_ref[...], w_ref[...], preferred_element_type=jnp.float32)
    o_ref[...] = y + b_ref[...][None, :] + acc_ref[0] + acc_ref[1]


def _post_final(h, W_root, b, acc):
    N = h.shape[0]
    Dout = W_root.shape[1]
    return pl.pallas_call(
        _post_final_body,
        out_shape=jax.ShapeDtypeStruct((N, Dout), jnp.float32),
    )(h, W_root, b, acc)


def kernel(x, edge_index, edge_type, W1_rel, W1_root, b1, g1, be1,
           W2_rel, W2_root, b2, g2, be2, W3_rel, W3_root, b3):
    N, _ = x.shape
    R = W1_rel.shape[0]
    E = edge_type.shape[0]
    CPW = -(-E // (NW * CHUNK))
    CPW = -(-CPW // 16) * 16
    EP = NW * CPW * CHUNK
    pad = EP - E

    src = edge_index[0]
    dst = edge_index[1]
    zpad = jnp.zeros((pad,), jnp.int32)
    src3 = jnp.concatenate([src, zpad]).reshape(NW, CPW, CHUNK)
    dst3 = jnp.concatenate([dst, zpad]).reshape(NW, CPW, CHUNK)
    et3 = jnp.concatenate(
        [edge_type, jnp.full((pad,), ETPAD, jnp.int32)]).reshape(NW, CPW, CHUNK)

    g3, w3 = _make_prep(N, R, CPW)(src3, et3, dst3)

    def layer(h, W_rel, W_root, b, post):
        Dout = W_rel.shape[2]
        T = _ttable(h, W_rel)
        acc = _make_edge_pass(N, Dout, CPW)(T, dst3, g3, w3)
        return post(h, W_root, b, acc)

    h = layer(x, W1_rel, W1_root, b1,
              lambda h_, w_, b_, a_: _post_bn(h_, w_, b_, a_, g1, be1))
    h = layer(h, W2_rel, W2_root, b2,
              lambda h_, w_, b_, a_: _post_bn(h_, w_, b_, a_, g2, be2))

    DP = 128
    W3p = jnp.pad(W3_rel, ((0, 0), (0, 0), (0, DP - W3_rel.shape[2])))
    W3rootp = jnp.pad(W3_root, ((0, 0), (0, DP - W3_root.shape[1])))
    b3p = jnp.pad(b3, (0, DP - b3.shape[0]))
    out16 = layer(h, W3p, W3rootp, b3p, _post_final)
    return out16[:, :W3_rel.shape[2]]
